# baseline (speedup 1.0000x reference)
import jax
import jax.numpy as jnp
from jax.experimental import pallas as pl
from jax.experimental.pallas import tpu as pltpu

_BM = 400
_NBUF = 2
_NSPLIT = 2
_SUB = _BM // _NSPLIT


def _gcn_body(x_ref, adj_hbm, wt_ref, b_ref, out_ref, buf_ref, support_ref,
              acc_ref, sems):
    nblk = adj_hbm.shape[0] // _BM

    def _copies(blk, slot):
        return [
            pltpu.make_async_copy(
                adj_hbm.at[pl.ds(blk * _BM + s * _SUB, _SUB), :],
                buf_ref.at[slot, pl.ds(s * _SUB, _SUB), :],
                sems.at[slot, s],
            )
            for s in range(_NSPLIT)
        ]

    def _start(blk, slot):
        for c in _copies(blk, slot):
            c.start()

    def _wait(blk, slot):
        for c in _copies(blk, slot):
            c.wait()

    for w in range(_NBUF):
        _start(w, w)

    support_ref[...] = jax.lax.dot_general(
        x_ref[...],
        wt_ref[...],
        (((1,), (1,)), ((), ())),
        preferred_element_type=jnp.float32,
    )

    for i in range(nblk):
        slot = i % _NBUF
        _wait(i, slot)
        blk = (
            jax.lax.dot_general(
                buf_ref[slot],
                support_ref[...],
                (((1,), (0,)), ((), ())),
                preferred_element_type=jnp.float32,
            )
            + b_ref[...]
        )
        acc_ref[i * _BM:(i + 1) * _BM, :] = blk
        if i + _NBUF < nblk:
            _start(i + _NBUF, slot)

    out_ref[...] = acc_ref[...].T


def kernel(input, adj, weight, bias):
    n, k = input.shape
    m = adj.shape[0]
    f = weight.shape[1]

    out_t = pl.pallas_call(
        _gcn_body,
        in_specs=[
            pl.BlockSpec((n, k), lambda: (0, 0)),
            pl.BlockSpec(memory_space=pl.ANY),
            pl.BlockSpec((f, k), lambda: (0, 0)),
            pl.BlockSpec((1, f), lambda: (0, 0)),
        ],
        out_specs=pl.BlockSpec((f, m), lambda: (0, 0)),
        out_shape=jax.ShapeDtypeStruct((f, m), jnp.float32),
        scratch_shapes=[
            pltpu.VMEM((_NBUF, _BM, n), jnp.float32),
            pltpu.VMEM((n, f), jnp.float32),
            pltpu.VMEM((m, f), jnp.float32),
            pltpu.SemaphoreType.DMA((_NBUF, _NSPLIT)),
        ],
    )(input, adj, weight.T, bias.reshape(1, f))
    return out_t.T


# speedup vs baseline: 1.0615x; 1.0615x over previous
import jax
import jax.numpy as jnp
from jax.experimental import pallas as pl
from jax.experimental.pallas import tpu as pltpu

_BM = 200
_NBUF = 4
_SIZES = [40, 160] + [_BM] * 48 + [160, 40]
_OFFS = [sum(_SIZES[:j]) for j in range(len(_SIZES))]


def _gcn_body(x_ref, adj_hbm, wt_ref, b_ref, out_ref, buf_ref, support_ref,
              acc_ref, sems):
    def _copy(blk, slot):
        sz = _SIZES[blk]
        return pltpu.make_async_copy(
            adj_hbm.at[pl.ds(_OFFS[blk], sz), :],
            buf_ref.at[slot, pl.ds(0, sz), :],
            sems.at[slot],
        )

    for w in range(_NBUF):
        _copy(w, w).start()

    support_ref[...] = jax.lax.dot_general(
        x_ref[...],
        wt_ref[...],
        (((1,), (1,)), ((), ())),
        preferred_element_type=jnp.float32,
    )

    for i in range(len(_SIZES)):
        slot = i % _NBUF
        sz = _SIZES[i]
        _copy(i, slot).wait()
        blk = (
            jax.lax.dot_general(
                buf_ref[slot, pl.ds(0, sz), :],
                support_ref[...],
                (((1,), (0,)), ((), ())),
                preferred_element_type=jnp.float32,
            )
            + b_ref[...]
        )
        acc_ref[pl.ds(_OFFS[i], sz), :] = blk
        if i + _NBUF < len(_SIZES):
            _copy(i + _NBUF, slot).start()

    out_ref[...] = acc_ref[...].T


def kernel(input, adj, weight, bias):
    n, k = input.shape
    m = adj.shape[0]
    f = weight.shape[1]

    out_t = pl.pallas_call(
        _gcn_body,
        in_specs=[
            pl.BlockSpec((n, k), lambda: (0, 0)),
            pl.BlockSpec(memory_space=pl.ANY),
            pl.BlockSpec((f, k), lambda: (0, 0)),
            pl.BlockSpec((1, f), lambda: (0, 0)),
        ],
        out_specs=pl.BlockSpec((f, m), lambda: (0, 0)),
        out_shape=jax.ShapeDtypeStruct((f, m), jnp.float32),
        scratch_shapes=[
            pltpu.VMEM((_NBUF, _BM, n), jnp.float32),
            pltpu.VMEM((n, f), jnp.float32),
            pltpu.VMEM((m, f), jnp.float32),
            pltpu.SemaphoreType.DMA((_NBUF,)),
        ],
    )(input, adj, weight.T, bias.reshape(1, f))
    return out_t.T
